# Initial kernel scaffold; baseline (speedup 1.0000x reference)
import jax
import jax.numpy as jnp
from jax.experimental import pallas as pl


def kernel(Za, Qa, Q, batch_seg):
    raise NotImplementedError("write your pallas kernel here")


# speedup vs baseline: 120.1180x; 120.1180x over previous
import functools

import jax
import jax.numpy as jnp
from jax import lax
from jax.experimental import pallas as pl
from jax.experimental.pallas import tpu as pltpu
from jax.experimental.pallas import tpu_sc as plsc

NC = 2
NS = 16
NW = NC * NS
L = 16


def _wid():
    return lax.axis_index("s") * NC + lax.axis_index("c")


def _pass1_body(M, T, B, seg_hbm, qa_hbm, sums_hbm, cnts_hbm,
                seg_v, qa_v, accs, accc, row_v):
    wid = _wid()
    base = wid * M
    lane_off = lax.iota(jnp.int32, L) * B
    ones = jnp.ones((L,), jnp.float32)
    zeros = jnp.zeros((L,), jnp.float32)

    def zero_body(j, _):
        accs[pl.ds(j * L, L)] = zeros
        accc[pl.ds(j * L, L)] = zeros
        return 0

    lax.fori_loop(0, (L * B) // L, zero_body, 0)

    def vec_body(v, _):
        sl = pl.ds(v * L, L)
        s = seg_v[sl]
        q = qa_v[sl]
        idx = lane_off + s
        plsc.addupdate_scatter(accs, [idx], q)
        plsc.addupdate_scatter(accc, [idx], ones)
        return 0

    for k in range(M // T):
        off = base + k * T
        pltpu.sync_copy(seg_hbm.at[pl.ds(off, T)], seg_v)
        pltpu.sync_copy(qa_hbm.at[pl.ds(off, T)], qa_v)
        lax.fori_loop(0, T // L, vec_body, 0)

    def red_sums(j, _):
        sl = pl.ds(j * L, L)
        tot = accs[sl]
        for i in range(1, L):
            tot = tot + accs[pl.ds(i * B + j * L, L)]
        row_v[sl] = tot
        return 0

    lax.fori_loop(0, B // L, red_sums, 0)
    pltpu.sync_copy(row_v, sums_hbm.at[pl.ds(wid * B, B)])

    def red_cnts(j, _):
        sl = pl.ds(j * L, L)
        tot = accc[sl]
        for i in range(1, L):
            tot = tot + accc[pl.ds(i * B + j * L, L)]
        row_v[sl] = tot
        return 0

    lax.fori_loop(0, B // L, red_cnts, 0)
    pltpu.sync_copy(row_v, cnts_hbm.at[pl.ds(wid * B, B)])


def _pass2_body(M, T, B, seg_hbm, qa_hbm, q_hbm, sums_hbm, cnts_hbm,
                out_hbm, rawq_hbm, seg_v, qa_v, sums_v, cnts_v,
                corr_v, qv_v, raw_v):
    wid = _wid()
    base = wid * M
    pltpu.sync_copy(sums_hbm, sums_v)
    pltpu.sync_copy(cnts_hbm, cnts_v)
    pltpu.sync_copy(q_hbm, qv_v)

    def comb_body(j, _):
        sl = pl.ds(j * L, L)
        s = sums_v[sl]
        c = cnts_v[sl]
        for i in range(1, NW):
            s = s + sums_v[pl.ds(i * B + j * L, L)]
            c = c + cnts_v[pl.ds(i * B + j * L, L)]
        raw_v[sl] = s
        corr_v[sl] = (qv_v[sl] - s) / c
        return 0

    lax.fori_loop(0, B // L, comb_body, 0)

    @pl.when(wid == 0)
    def _():
        pltpu.sync_copy(raw_v, rawq_hbm)

    def vec_body(v, _):
        sl = pl.ds(v * L, L)
        s = seg_v[sl]
        q = qa_v[sl]
        c = plsc.load_gather(corr_v, [s])
        qa_v[sl] = q + c
        return 0

    for k in range(M // T):
        off = base + k * T
        pltpu.sync_copy(seg_hbm.at[pl.ds(off, T)], seg_v)
        pltpu.sync_copy(qa_hbm.at[pl.ds(off, T)], qa_v)
        lax.fori_loop(0, T // L, vec_body, 0)
        pltpu.sync_copy(qa_v, out_hbm.at[pl.ds(off, T)])


def kernel(Za, Qa, Q, batch_seg):
    del Za
    N = Qa.shape[0]
    B = Q.shape[0]
    assert N % NW == 0
    M = N // NW
    T1 = 20000
    T2 = 10000
    assert M % T1 == 0 and M % T2 == 0

    seg = batch_seg.astype(jnp.int32)
    qa = Qa.astype(jnp.float32)

    mesh = plsc.VectorSubcoreMesh(core_axis_name="c", subcore_axis_name="s")

    pass1 = pl.kernel(
        functools.partial(_pass1_body, M, T1, B),
        out_type=(
            jax.ShapeDtypeStruct((NW * B,), jnp.float32),
            jax.ShapeDtypeStruct((NW * B,), jnp.float32),
        ),
        mesh=mesh,
        compiler_params=pltpu.CompilerParams(needs_layout_passes=False),
        scratch_types=[
            pltpu.VMEM((T1,), jnp.int32),
            pltpu.VMEM((T1,), jnp.float32),
            pltpu.VMEM((L * B,), jnp.float32),
            pltpu.VMEM((L * B,), jnp.float32),
            pltpu.VMEM((B,), jnp.float32),
        ],
    )
    sums, cnts = pass1(seg, qa)

    pass2 = pl.kernel(
        functools.partial(_pass2_body, M, T2, B),
        out_type=(
            jax.ShapeDtypeStruct((N,), jnp.float32),
            jax.ShapeDtypeStruct((B,), jnp.float32),
        ),
        mesh=mesh,
        compiler_params=pltpu.CompilerParams(needs_layout_passes=False),
        scratch_types=[
            pltpu.VMEM((T2,), jnp.int32),
            pltpu.VMEM((T2,), jnp.float32),
            pltpu.VMEM((NW * B,), jnp.float32),
            pltpu.VMEM((NW * B,), jnp.float32),
            pltpu.VMEM((B,), jnp.float32),
            pltpu.VMEM((B,), jnp.float32),
            pltpu.VMEM((B,), jnp.float32),
        ],
    )
    out, rawq = pass2(seg, qa, Q.astype(jnp.float32), sums, cnts)
    return (out, rawq)


# speedup vs baseline: 202.2175x; 1.6835x over previous
import functools

import jax
import jax.numpy as jnp
from jax import lax
from jax.experimental import pallas as pl
from jax.experimental.pallas import tpu as pltpu
from jax.experimental.pallas import tpu_sc as plsc

NC = 2
NS = 16
NW = NC * NS
L = 16


def _wid():
    return lax.axis_index("s") * NC + lax.axis_index("c")


def _pass1_body(M, T, B, BP, seg_hbm, qa_hbm, sums_hbm, cnts_hbm,
                seg_v, qa_v, accs, accc, row_v):
    wid = _wid()
    base = wid * M
    lane_off = lax.iota(jnp.int32, L) * BP
    ones = jnp.ones((L,), jnp.float32)
    zeros = jnp.zeros((L,), jnp.float32)

    def zero_body(j, _):
        accs[pl.ds(j * L, L)] = zeros
        accc[pl.ds(j * L, L)] = zeros
        return 0

    lax.fori_loop(0, (L * BP + L - 1) // L, zero_body, 0)

    def vec_body(v, _):
        sl = pl.ds(v * L, L)
        s = seg_v[sl]
        q = qa_v[sl]
        idx = lane_off + s
        plsc.addupdate_scatter(accs, [idx], q)
        plsc.addupdate_scatter(accc, [idx], ones)
        return 0

    for k in range(M // T):
        off = base + k * T
        pltpu.sync_copy(seg_hbm.at[pl.ds(off, T)], seg_v)
        pltpu.sync_copy(qa_hbm.at[pl.ds(off, T)], qa_v)
        lax.fori_loop(0, T // L, vec_body, 0)

    def red_sums(j, _):
        sl = pl.ds(j * L, L)
        tot = accs[sl]
        for i in range(1, L):
            tot = tot + accs[pl.ds(i * BP + j * L, L)]
        row_v[sl] = tot
        return 0

    lax.fori_loop(0, B // L, red_sums, 0)
    pltpu.sync_copy(row_v, sums_hbm.at[pl.ds(wid * B, B)])

    def red_cnts(j, _):
        sl = pl.ds(j * L, L)
        tot = accc[sl]
        for i in range(1, L):
            tot = tot + accc[pl.ds(i * BP + j * L, L)]
        row_v[sl] = tot
        return 0

    lax.fori_loop(0, B // L, red_cnts, 0)
    pltpu.sync_copy(row_v, cnts_hbm.at[pl.ds(wid * B, B)])


def _pass2_body(M, T, B, seg_hbm, qa_hbm, q_hbm, sums_hbm, cnts_hbm,
                out_hbm, rawq_hbm, seg_v, qa_v, sums_v, cnts_v,
                corr_v, qv_v, raw_v):
    wid = _wid()
    base = wid * M
    pltpu.sync_copy(sums_hbm, sums_v)
    pltpu.sync_copy(cnts_hbm, cnts_v)
    pltpu.sync_copy(q_hbm, qv_v)

    def comb_body(j, _):
        sl = pl.ds(j * L, L)
        s = sums_v[sl]
        c = cnts_v[sl]
        for i in range(1, NW):
            s = s + sums_v[pl.ds(i * B + j * L, L)]
            c = c + cnts_v[pl.ds(i * B + j * L, L)]
        raw_v[sl] = s
        corr_v[sl] = (qv_v[sl] - s) / c
        return 0

    lax.fori_loop(0, B // L, comb_body, 0)

    @pl.when(wid == 0)
    def _():
        pltpu.sync_copy(raw_v, rawq_hbm)

    def vec_body(v, _):
        sl = pl.ds(v * L, L)
        s = seg_v[sl]
        q = qa_v[sl]
        c = plsc.load_gather(corr_v, [s])
        qa_v[sl] = q + c
        return 0

    for k in range(M // T):
        off = base + k * T
        pltpu.sync_copy(seg_hbm.at[pl.ds(off, T)], seg_v)
        pltpu.sync_copy(qa_hbm.at[pl.ds(off, T)], qa_v)
        lax.fori_loop(0, T // L, vec_body, 0)
        pltpu.sync_copy(qa_v, out_hbm.at[pl.ds(off, T)])


def kernel(Za, Qa, Q, batch_seg):
    del Za
    N = Qa.shape[0]
    B = Q.shape[0]
    assert N % NW == 0
    M = N // NW
    T1 = 20000
    T2 = 10000
    assert M % T1 == 0 and M % T2 == 0

    seg = batch_seg.astype(jnp.int32)
    qa = Qa.astype(jnp.float32)

    mesh = plsc.VectorSubcoreMesh(core_axis_name="c", subcore_axis_name="s")

    BP = B + 1
    pass1 = pl.kernel(
        functools.partial(_pass1_body, M, T1, B, BP),
        out_type=(
            jax.ShapeDtypeStruct((NW * B,), jnp.float32),
            jax.ShapeDtypeStruct((NW * B,), jnp.float32),
        ),
        mesh=mesh,
        compiler_params=pltpu.CompilerParams(needs_layout_passes=False),
        scratch_types=[
            pltpu.VMEM((T1,), jnp.int32),
            pltpu.VMEM((T1,), jnp.float32),
            pltpu.VMEM((L * BP,), jnp.float32),
            pltpu.VMEM((L * BP,), jnp.float32),
            pltpu.VMEM((B,), jnp.float32),
        ],
    )
    sums, cnts = pass1(seg, qa)

    pass2 = pl.kernel(
        functools.partial(_pass2_body, M, T2, B),
        out_type=(
            jax.ShapeDtypeStruct((N,), jnp.float32),
            jax.ShapeDtypeStruct((B,), jnp.float32),
        ),
        mesh=mesh,
        compiler_params=pltpu.CompilerParams(needs_layout_passes=False),
        scratch_types=[
            pltpu.VMEM((T2,), jnp.int32),
            pltpu.VMEM((T2,), jnp.float32),
            pltpu.VMEM((NW * B,), jnp.float32),
            pltpu.VMEM((NW * B,), jnp.float32),
            pltpu.VMEM((B,), jnp.float32),
            pltpu.VMEM((B,), jnp.float32),
            pltpu.VMEM((B,), jnp.float32),
        ],
    )
    out, rawq = pass2(seg, qa, Q.astype(jnp.float32), sums, cnts)
    return (out, rawq)


# speedup vs baseline: 262.4645x; 1.2979x over previous
import functools

import jax
import jax.numpy as jnp
from jax import lax
from jax.experimental import pallas as pl
from jax.experimental.pallas import tpu as pltpu
from jax.experimental.pallas import tpu_sc as plsc

NC = 2
NS = 16
NW = NC * NS
L = 16
U = 5


def _wid():
    return lax.axis_index("s") * NC + lax.axis_index("c")


def _pass1_body(M, T, B, BP, seg_hbm, qa_hbm, sums_hbm, cnts_hbm,
                seg0, seg1, qa0, qa1, accs, accc, row_v, sems):
    wid = _wid()
    base = wid * M
    lane_off = lax.iota(jnp.int32, L) * BP
    ones = jnp.ones((L,), jnp.float32)
    zeros = jnp.zeros((L,), jnp.float32)
    bufs = ((seg0, qa0), (seg1, qa1))
    nchunks = M // T

    def issue(k, slot):
        off = base + k * T
        sb, qb = bufs[slot]
        c1 = pltpu.async_copy(seg_hbm.at[pl.ds(off, T)], sb, sems.at[slot])
        c2 = pltpu.async_copy(qa_hbm.at[pl.ds(off, T)], qb, sems.at[slot])
        return (c1, c2)

    copies = [issue(0, 0), None]

    def zero_body(j, _):
        for u in range(U):
            sl = pl.ds((j * U + u) * L, L)
            accs[sl] = zeros
            accc[sl] = zeros
        return 0

    lax.fori_loop(0, (L * BP) // (L * U), zero_body, 0)

    for k in range(nchunks):
        slot = k % 2
        if k + 1 < nchunks:
            copies[(k + 1) % 2] = issue(k + 1, (k + 1) % 2)
        for c in copies[slot]:
            c.wait()
        sb, qb = bufs[slot]

        def vec_body(v, _):
            for u in range(U):
                sl = pl.ds((v * U + u) * L, L)
                s = sb[sl]
                q = qb[sl]
                idx = lane_off + s
                plsc.addupdate_scatter(accs, [idx], q)
                plsc.addupdate_scatter(accc, [idx], ones)
            return 0

        lax.fori_loop(0, T // (L * U), vec_body, 0)

    def red_sums(j, _):
        sl = pl.ds(j * L, L)
        tot = accs[sl]
        for i in range(1, L):
            tot = tot + accs[pl.ds(i * BP + j * L, L)]
        row_v[sl] = tot
        return 0

    lax.fori_loop(0, B // L, red_sums, 0)
    pltpu.sync_copy(row_v, sums_hbm.at[pl.ds(wid * B, B)])

    def red_cnts(j, _):
        sl = pl.ds(j * L, L)
        tot = accc[sl]
        for i in range(1, L):
            tot = tot + accc[pl.ds(i * BP + j * L, L)]
        row_v[sl] = tot
        return 0

    lax.fori_loop(0, B // L, red_cnts, 0)
    pltpu.sync_copy(row_v, cnts_hbm.at[pl.ds(wid * B, B)])


def _pass2_body(M, T, B, seg_hbm, qa_hbm, q_hbm, sums_hbm, cnts_hbm,
                out_hbm, rawq_hbm, seg0, seg1, qa0, qa1, out0, out1,
                big_v, corr_v, qv_v, raw_v, sems, osems):
    wid = _wid()
    base = wid * M
    bufs = ((seg0, qa0, out0), (seg1, qa1, out1))
    nchunks = M // T

    def issue(k, slot):
        off = base + k * T
        sb, qb, _ = bufs[slot]
        c1 = pltpu.async_copy(seg_hbm.at[pl.ds(off, T)], sb, sems.at[slot])
        c2 = pltpu.async_copy(qa_hbm.at[pl.ds(off, T)], qb, sems.at[slot])
        return (c1, c2)

    copies = [issue(0, 0), None]

    pltpu.sync_copy(q_hbm, qv_v)
    pltpu.sync_copy(sums_hbm, big_v)

    def comb_sums(j, _):
        sl = pl.ds(j * L, L)
        s = big_v[sl]
        for i in range(1, NW):
            s = s + big_v[pl.ds(i * B + j * L, L)]
        raw_v[sl] = s
        return 0

    lax.fori_loop(0, B // L, comb_sums, 0)
    pltpu.sync_copy(cnts_hbm, big_v)

    def comb_cnts(j, _):
        sl = pl.ds(j * L, L)
        c = big_v[sl]
        for i in range(1, NW):
            c = c + big_v[pl.ds(i * B + j * L, L)]
        corr_v[sl] = (qv_v[sl] - raw_v[sl]) / c
        return 0

    lax.fori_loop(0, B // L, comb_cnts, 0)

    @pl.when(wid == 0)
    def _():
        pltpu.sync_copy(raw_v, rawq_hbm)

    outcp = [None, None]
    for k in range(nchunks):
        slot = k % 2
        if k + 1 < nchunks:
            copies[(k + 1) % 2] = issue(k + 1, (k + 1) % 2)
        for c in copies[slot]:
            c.wait()
        if outcp[slot] is not None:
            outcp[slot].wait()
        sb, qb, ob = bufs[slot]

        def vec_body(v, _):
            for u in range(U):
                sl = pl.ds((v * U + u) * L, L)
                s = sb[sl]
                q = qb[sl]
                c = plsc.load_gather(corr_v, [s])
                ob[sl] = q + c
            return 0

        lax.fori_loop(0, T // (L * U), vec_body, 0)
        off = base + k * T
        outcp[slot] = pltpu.async_copy(ob, out_hbm.at[pl.ds(off, T)],
                                       osems.at[slot])
    for cp in outcp:
        if cp is not None:
            cp.wait()


def kernel(Za, Qa, Q, batch_seg):
    del Za
    N = Qa.shape[0]
    B = Q.shape[0]
    assert N % NW == 0
    M = N // NW
    T = 10000
    assert M % T == 0 and (T // L) % U == 0

    seg = batch_seg.astype(jnp.int32)
    qa = Qa.astype(jnp.float32)

    BP = B + 1

    mesh = plsc.VectorSubcoreMesh(core_axis_name="c", subcore_axis_name="s")

    pass1 = pl.kernel(
        functools.partial(_pass1_body, M, T, B, BP),
        out_type=(
            jax.ShapeDtypeStruct((NW * B,), jnp.float32),
            jax.ShapeDtypeStruct((NW * B,), jnp.float32),
        ),
        mesh=mesh,
        compiler_params=pltpu.CompilerParams(needs_layout_passes=False),
        scratch_types=[
            pltpu.VMEM((T,), jnp.int32),
            pltpu.VMEM((T,), jnp.int32),
            pltpu.VMEM((T,), jnp.float32),
            pltpu.VMEM((T,), jnp.float32),
            pltpu.VMEM((L * BP,), jnp.float32),
            pltpu.VMEM((L * BP,), jnp.float32),
            pltpu.VMEM((B,), jnp.float32),
            pltpu.SemaphoreType.DMA((2,)),
        ],
    )
    sums, cnts = pass1(seg, qa)

    pass2 = pl.kernel(
        functools.partial(_pass2_body, M, T, B),
        out_type=(
            jax.ShapeDtypeStruct((N,), jnp.float32),
            jax.ShapeDtypeStruct((B,), jnp.float32),
        ),
        mesh=mesh,
        compiler_params=pltpu.CompilerParams(needs_layout_passes=False),
        scratch_types=[
            pltpu.VMEM((T,), jnp.int32),
            pltpu.VMEM((T,), jnp.int32),
            pltpu.VMEM((T,), jnp.float32),
            pltpu.VMEM((T,), jnp.float32),
            pltpu.VMEM((T,), jnp.float32),
            pltpu.VMEM((T,), jnp.float32),
            pltpu.VMEM((NW * B,), jnp.float32),
            pltpu.VMEM((B,), jnp.float32),
            pltpu.VMEM((B,), jnp.float32),
            pltpu.VMEM((B,), jnp.float32),
            pltpu.SemaphoreType.DMA((2,)),
            pltpu.SemaphoreType.DMA((2,)),
        ],
    )
    out, rawq = pass2(seg, qa, Q.astype(jnp.float32), sums, cnts)
    return (out, rawq)
